# baseline (speedup 1.0000x reference)
import numpy as np
import jax
import jax.numpy as jnp
from jax.experimental import pallas as pl
from jax.experimental.pallas import tpu as pltpu


def _lane_matrix(w: int) -> np.ndarray:
    m = np.zeros((w, 2 * w), np.float32)
    j = np.arange(w)
    m[np.maximum(j - 1, 0), 2 * j] += 0.25
    m[j, 2 * j] += 0.75
    m[j, 2 * j + 1] += 0.75
    m[np.minimum(j + 1, w - 1), 2 * j + 1] += 0.25
    return m


def _pair_matrix(w: int) -> np.ndarray:
    b = _lane_matrix(w)
    m = np.zeros((2 * w, 4 * w), np.float32)
    m[:w, : 2 * w] = b
    m[w:, 2 * w:] = b
    return m


def _up2x_kernel(x_ref, bt2_ref, o_ref):
    _, hw, t = x_ref.shape
    wp = bt2_ref.shape[0]
    w2 = bt2_ref.shape[1] // 2
    hp = hw // wp
    xt = x_ref[0].T
    y = jnp.dot(xt.reshape(t * hp, wp), bt2_ref[...],
                preferred_element_type=jnp.float32).reshape(t, hp, 2 * w2)
    e = y[:, :, :w2]
    o = y[:, :, w2:]
    o_up = jnp.concatenate([e[:, :1], o[:, :-1]], axis=1)
    e_dn = jnp.concatenate([e[:, 1:], o[:, -1:]], axis=1)
    o_ref[:, 0::4, :] = 0.25 * o_up + 0.75 * e
    o_ref[:, 1::4, :] = 0.75 * e + 0.25 * o
    o_ref[:, 2::4, :] = 0.25 * e + 0.75 * o
    o_ref[:, 3::4, :] = 0.75 * o + 0.25 * e_dn


def kernel(x):
    n, c, h, w = map(int, x.shape)
    ctile = 256
    bt2 = jnp.asarray(_pair_matrix(w))
    out = pl.pallas_call(
        _up2x_kernel,
        out_shape=jax.ShapeDtypeStruct((n * c, 2 * h, 2 * w), x.dtype),
        grid=(n, c // ctile),
        in_specs=[pl.BlockSpec((1, h * w, ctile),
                               lambda i, j: (i, 0, j)),
                  pl.BlockSpec((2 * w, 4 * w), lambda i, j: (0, 0))],
        out_specs=pl.BlockSpec((ctile, 2 * h, 2 * w),
                               lambda i, j, _c=c // ctile: (i * _c + j, 0, 0)),
        compiler_params=pltpu.CompilerParams(
            dimension_semantics=("parallel", "parallel"),
            vmem_limit_bytes=56 * 1024 * 1024,
        ),
    )(jnp.transpose(x.reshape(n, c, h * w), (0, 2, 1)), bt2)
    return out.reshape(n, c, 2 * h, 2 * w)
